# baseline (speedup 1.0000x reference)
import functools
import jax
import jax.numpy as jnp
from jax import lax
from jax.experimental import pallas as pl
from jax.experimental.pallas import tpu as pltpu
from jax.experimental.pallas import tpu_sc as plsc

N = 10000
E = 320000
G = 64
H = 10
NW = 32
EPW = E // NW
BE = 3200
BN = 2000
F32 = jnp.float32
I32 = jnp.int32
TRT = (((1,), (1,)), ((), ()))
HI = lax.Precision.HIGHEST


def _a1_body(e_ref, w_ref, out_ref):
    out_ref[...] = lax.dot_general(w_ref[...], e_ref[...], TRT, precision=HI,
                                   preferred_element_type=F32)


def _a2_body(x_ref, w_ref, out_ref):
    out_ref[...] = lax.dot_general(w_ref[...], x_ref[...], TRT, precision=HI,
                                   preferred_element_type=F32)


def _b_body(baseT, xallT, srch, dsth, eidxh, wth, wgbh, gtfh,
            eo, e2np, e2gp,
            src_v, dst_v, eidx_v, acc_v,
            base0, base1, xs0, xs1, xd0, xd1,
            e2n_v, e2g_v, wt_v, wgb_v, gt_v, gp_v, sem0, sem1):
    wid = lax.axis_index("s") * 2 + lax.axis_index("c")
    be = wid * EPW
    pltpu.sync_copy(srch.at[pl.ds(be, EPW)], src_v)
    pltpu.sync_copy(dsth.at[pl.ds(be, EPW)], dst_v)
    pltpu.sync_copy(eidxh.at[pl.ds(be, EPW)], eidx_v)
    pltpu.sync_copy(wth, wt_v)
    pltpu.sync_copy(wgbh, wgb_v)
    pltpu.sync_copy(gtfh, gt_v)
    nch = EPW // 16

    def gpbody(d, c):
        for cc in range(G // 16):
            a = wt_v[16 + d, :]
            for k in range(32):
                a = a + gt_v[pl.ds(k * G + cc * 16, 16)] * wgb_v[d * 32 + k, :]
            gp_v[pl.ds(d * G + cc * 16, 16)] = a
        return c
    lax.fori_loop(0, H, gpbody, 0)

    b_vec = wt_v[H, :]

    def zacc(i, c):
        acc_v[pl.ds(i * 16, 16)] = b_vec
        return c
    lax.fori_loop(0, nch, zacc, 0)

    zero = jnp.zeros((16,), F32)

    def zn(i, c):
        e2n_v[pl.ds(i * 16, 16)] = zero
        return c
    lax.fori_loop(0, N // 16, zn, 0)
    for j in range(G // 16):
        e2g_v[pl.ds(j * 16, 16)] = zero

    bufs = [(base0, xs0, xd0, sem0), (base1, xs1, xd1, sem1)]

    def start(d):
        bb, b1, b2, sm = bufs[d % 2]
        return [pltpu.async_copy(baseT.at[d, pl.ds(be, EPW)], bb, sm),
                pltpu.async_copy(xallT.at[d], b1, sm),
                pltpu.async_copy(xallT.at[H + d], b2, sm)]

    cps = start(0)
    for d in range(H):
        for c in cps:
            c.wait()
        if d < H - 1:
            cps = start(d + 1)
        base_v, xs_v, xd_v, _ = bufs[d % 2]
        w_vec = wt_v[d, :]
        off = jnp.full((16,), d * G, I32)

        def ebody(i, c):
            ds_ = pl.ds(i * 16, 16)
            hh = (base_v[ds_]
                  + plsc.load_gather(xs_v, [src_v[ds_]])
                  + plsc.load_gather(xd_v, [dst_v[ds_]])
                  + plsc.load_gather(gp_v, [eidx_v[ds_] + off]))
            acc_v[ds_] = acc_v[ds_] + jnp.maximum(hh, 0.0) * w_vec
            return c
        lax.fori_loop(0, nch, ebody, 0)

    pltpu.sync_copy(acc_v, eo.at[pl.ds(be, EPW)])

    def sbody(i, c):
        ds_ = pl.ds(i * 16, 16)
        v = acc_v[ds_]
        plsc.addupdate_scatter(e2n_v, [dst_v[ds_]], v)
        plsc.addupdate_scatter(e2g_v, [eidx_v[ds_]], v)
        return c
    lax.fori_loop(0, nch, sbody, 0)

    pltpu.sync_copy(e2n_v, e2np.at[wid])
    pltpu.sync_copy(e2g_v, e2gp.at[wid])


def _c_body(x_ref, gT_ref, parts_ref, nrow_ref,
            wnx_ref, bn_ref, wng_ref, wcol_ref, lng_ref, lnb_ref, wn2_ref,
            nout_ref, n2g_ref):
    h = lax.dot_general(wnx_ref[...], x_ref[...], TRT, precision=HI,
                        preferred_element_type=F32) + bn_ref[...]
    gnT = jnp.dot(wng_ref[...], gT_ref[...], precision=HI, preferred_element_type=F32)
    ohg = (lax.broadcasted_iota(I32, (G, N), 0).astype(F32)
           == nrow_ref[...]).astype(F32)
    h = h + jnp.dot(gnT, ohg, precision=HI, preferred_element_type=F32)
    e2n_row = jnp.sum(parts_ref[...], axis=0, keepdims=True)
    h = h + wcol_ref[...] * e2n_row
    mu = jnp.mean(h, axis=0, keepdims=True)
    d0 = h - mu
    var = jnp.mean(d0 * d0, axis=0, keepdims=True)
    y = d0 * lax.rsqrt(var + 1e-5) * lng_ref[...] + lnb_ref[...]
    r = jnp.maximum(y, 0.0)
    r1 = jnp.concatenate([r, jnp.ones((1, N), F32)], axis=0)
    no = jnp.dot(wn2_ref[...], r1, precision=HI, preferred_element_type=F32)
    nout_ref[...] = no
    n2g_ref[...] = lax.dot_general(no, ohg, TRT, precision=HI, preferred_element_type=F32)


def _d_body(gT_ref, n2g_ref, e2gp_ref, wg1_ref, wg2_ref, wg3_ref, out_ref):
    e2g_row = jnp.sum(e2gp_ref[...], axis=0, keepdims=True)
    ones = jnp.ones((1, G), F32)
    gin = jnp.concatenate([gT_ref[...], n2g_ref[...], e2g_row, ones], axis=0)
    h1 = jnp.maximum(jnp.dot(wg1_ref[...], gin, preferred_element_type=F32), 0.0)
    h1 = jnp.concatenate([h1, ones], axis=0)
    h2 = jnp.maximum(jnp.dot(wg2_ref[...], h1, preferred_element_type=F32), 0.0)
    h2 = jnp.concatenate([h2, ones], axis=0)
    out_ref[...] = jnp.dot(wg3_ref[...], h2, preferred_element_type=F32)


@jax.jit
def kernel(x, e, g, edges, node_idx, edge_idx, W_e1, b_e1, W_e2, b_e2,
           W_n1, b_n1, ln_g, ln_b, W_n2, b_n2,
           W_g1, b_g1, W_g2, b_g2, W_g3, b_g3):
    src = edges[0]
    dst = edges[1]

    gT = g.T
    nrow = node_idx.astype(F32)[None, :]

    WE = jnp.zeros((16, 16), F32).at[:H].set(W_e1[:16].T)
    WA2 = jnp.concatenate([W_e1[16:144].T, W_e1[144:272].T], axis=0)
    WA2 = jnp.pad(WA2, ((0, 4), (0, 0)))
    WNX = W_n1[:128].T
    bn_col = b_n1[:, None]
    WNG = W_n1[128:160].T
    wcol = W_n1[160][:, None]
    lng = ln_g[:, None]
    lnb = ln_b[:, None]
    WN2 = jnp.concatenate([W_n2.T, b_n2[None, :]], axis=1)
    WG1 = jnp.concatenate([W_g1.T, b_g1[:, None]], axis=1)
    WG2 = jnp.concatenate([W_g2.T, b_g2[:, None]], axis=1)
    WG3 = jnp.concatenate([W_g3.T, b_g3[:, None]], axis=1)
    wtab = jnp.zeros((32,), F32).at[:H].set(W_e2[:, 0]).at[H].set(b_e2[0])
    wtab = wtab.at[16:16 + H].set(b_e1)
    wtab = jnp.tile(wtab[:, None], (1, 16))
    wgb = jnp.tile(W_e1[272:304].T.reshape(-1)[:, None], (1, 16))
    gtf = gT.reshape(-1)

    baseT = pl.pallas_call(
        _a1_body,
        grid=(E // BE,),
        in_specs=[
            pl.BlockSpec((BE, 16), lambda i: (i, 0)),
            pl.BlockSpec((16, 16), lambda i: (0, 0)),
        ],
        out_specs=pl.BlockSpec((16, BE), lambda i: (0, i)),
        out_shape=jax.ShapeDtypeStruct((16, E), F32),
    )(e, WE)

    xallT = pl.pallas_call(
        _a2_body,
        grid=(1,),
        in_specs=[
            pl.BlockSpec((N, 128), lambda i: (0, 0)),
            pl.BlockSpec((24, 128), lambda i: (0, 0)),
        ],
        out_specs=pl.BlockSpec((24, N), lambda i: (0, 0)),
        out_shape=jax.ShapeDtypeStruct((24, N), F32),
    )(x, WA2)

    mesh = plsc.VectorSubcoreMesh(core_axis_name="c", subcore_axis_name="s")
    scb = functools.partial(
        pl.kernel,
        out_type=[
            jax.ShapeDtypeStruct((E,), F32),
            jax.ShapeDtypeStruct((NW, N), F32),
            jax.ShapeDtypeStruct((NW, G), F32),
        ],
        mesh=mesh,
        compiler_params=pltpu.CompilerParams(
            use_tc_tiling_on_sc=False, needs_layout_passes=False),
        scratch_types=[
            pltpu.VMEM((EPW,), I32),
            pltpu.VMEM((EPW,), I32),
            pltpu.VMEM((EPW,), I32),
            pltpu.VMEM((EPW,), F32),
            pltpu.VMEM((EPW,), F32),
            pltpu.VMEM((EPW,), F32),
            pltpu.VMEM((N,), F32),
            pltpu.VMEM((N,), F32),
            pltpu.VMEM((N,), F32),
            pltpu.VMEM((N,), F32),
            pltpu.VMEM((N,), F32),
            pltpu.VMEM((G,), F32),
            pltpu.VMEM((32, 16), F32),
            pltpu.VMEM((320, 16), F32),
            pltpu.VMEM((2048,), F32),
            pltpu.VMEM((H * G,), F32),
            pltpu.SemaphoreType.DMA,
            pltpu.SemaphoreType.DMA,
        ],
    )
    edge_out_flat, e2n_parts, e2g_parts = scb(_b_body)(
        baseT, xallT, src, dst, edge_idx, wtab, wgb, gtf)

    node_outT, n2g = pl.pallas_call(
        _c_body,
        grid=(1,),
        in_specs=[
            pl.BlockSpec((N, 128), lambda i: (0, 0)),
            pl.BlockSpec((32, G), lambda i: (0, 0)),
            pl.BlockSpec((NW, N), lambda i: (0, 0)),
            pl.BlockSpec((1, N), lambda i: (0, 0)),
            pl.BlockSpec((H, 128), lambda i: (0, 0)),
            pl.BlockSpec((H, 1), lambda i: (0, 0)),
            pl.BlockSpec((H, 32), lambda i: (0, 0)),
            pl.BlockSpec((H, 1), lambda i: (0, 0)),
            pl.BlockSpec((H, 1), lambda i: (0, 0)),
            pl.BlockSpec((H, 1), lambda i: (0, 0)),
            pl.BlockSpec((1, H + 1), lambda i: (0, 0)),
        ],
        out_specs=[
            pl.BlockSpec((1, N), lambda i: (0, 0)),
            pl.BlockSpec((1, G), lambda i: (0, 0)),
        ],
        out_shape=[
            jax.ShapeDtypeStruct((1, N), F32),
            jax.ShapeDtypeStruct((1, G), F32),
        ],
    )(x, gT, e2n_parts, nrow, WNX, bn_col, WNG, wcol, lng, lnb, WN2)

    globT = pl.pallas_call(
        _d_body,
        grid=(1,),
        in_specs=[
            pl.BlockSpec((32, G), lambda i: (0, 0)),
            pl.BlockSpec((1, G), lambda i: (0, 0)),
            pl.BlockSpec((NW, G), lambda i: (0, 0)),
            pl.BlockSpec((H, 35), lambda i: (0, 0)),
            pl.BlockSpec((H, H + 1), lambda i: (0, 0)),
            pl.BlockSpec((1, H + 1), lambda i: (0, 0)),
        ],
        out_specs=pl.BlockSpec((1, G), lambda i: (0, 0)),
        out_shape=jax.ShapeDtypeStruct((1, G), F32),
    )(gT, n2g, e2g_parts, WG1, WG2, WG3)

    edge_out = edge_out_flat[:, None]
    node_out = node_outT.reshape(N, 1)
    glob_out = globT.reshape(G, 1)
    return (edge_out, node_out, glob_out)


# speedup vs baseline: 1.8265x; 1.8265x over previous
import functools
import jax
import jax.numpy as jnp
from jax import lax
from jax.experimental import pallas as pl
from jax.experimental.pallas import tpu as pltpu
from jax.experimental.pallas import tpu_sc as plsc

N = 10000
E = 320000
G = 64
H = 10
NW = 32
EPW = E // NW
BE = 6400
BN = 2000
F32 = jnp.float32
I32 = jnp.int32
TRT = (((1,), (1,)), ((), ()))
HI = lax.Precision.HIGHEST


def _a1_body(eT_ref, w_ref, out_ref):
    out_ref[...] = jnp.dot(w_ref[...], eT_ref[...], preferred_element_type=F32)


def _a2_body(x_ref, w_ref, out_ref):
    out_ref[...] = lax.dot_general(w_ref[...], x_ref[...], TRT, precision=HI,
                                   preferred_element_type=F32)


def _b_body(baseT, xallT, srch, dsth, eidxh, wth, wgbh, gtfh,
            eo, e2np, e2gp,
            src_v, dst_v, eidx_v, acc_v,
            base0, base1, xs0, xs1, xd0, xd1,
            e2n_v, e2g_v, wt_v, wgb_v, gt_v, gp_v, sem0, sem1):
    wid = lax.axis_index("s") * 2 + lax.axis_index("c")
    be = wid * EPW
    pltpu.sync_copy(srch.at[pl.ds(be, EPW)], src_v)
    pltpu.sync_copy(dsth.at[pl.ds(be, EPW)], dst_v)
    pltpu.sync_copy(eidxh.at[pl.ds(be, EPW)], eidx_v)
    pltpu.sync_copy(wth, wt_v)
    pltpu.sync_copy(wgbh, wgb_v)
    pltpu.sync_copy(gtfh, gt_v)
    nch = EPW // 16

    def gpbody(d, c):
        for cc in range(G // 16):
            a = wt_v[16 + d, :]
            for k in range(32):
                a = a + gt_v[pl.ds(k * G + cc * 16, 16)] * wgb_v[d * 32 + k, :]
            gp_v[pl.ds(d * G + cc * 16, 16)] = a
        return c
    lax.fori_loop(0, H, gpbody, 0)

    b_vec = wt_v[H, :]

    def zacc(i, c):
        acc_v[pl.ds(i * 16, 16)] = b_vec
        return c
    lax.fori_loop(0, nch, zacc, 0)

    zero = jnp.zeros((16,), F32)

    def zn(i, c):
        e2n_v[pl.ds(i * 16, 16)] = zero
        return c
    lax.fori_loop(0, N // 16, zn, 0)
    for j in range(G // 16):
        e2g_v[pl.ds(j * 16, 16)] = zero

    bufs = [(base0, xs0, xd0, sem0), (base1, xs1, xd1, sem1)]

    def start(d):
        bb, b1, b2, sm = bufs[d % 2]
        return [pltpu.async_copy(baseT.at[d, pl.ds(be, EPW)], bb, sm),
                pltpu.async_copy(xallT.at[d], b1, sm),
                pltpu.async_copy(xallT.at[H + d], b2, sm)]

    cps = start(0)
    for d in range(H):
        for c in cps:
            c.wait()
        if d < H - 1:
            cps = start(d + 1)
        base_v, xs_v, xd_v, _ = bufs[d % 2]
        w_vec = wt_v[d, :]
        off = jnp.full((16,), d * G, I32)

        def ebody(i, c):
            ds_ = pl.ds(i * 16, 16)
            hh = (base_v[ds_]
                  + plsc.load_gather(xs_v, [src_v[ds_]])
                  + plsc.load_gather(xd_v, [dst_v[ds_]])
                  + plsc.load_gather(gp_v, [eidx_v[ds_] + off]))
            acc_v[ds_] = acc_v[ds_] + jnp.maximum(hh, 0.0) * w_vec
            return c
        lax.fori_loop(0, nch, ebody, 0)

    pltpu.sync_copy(acc_v, eo.at[pl.ds(be, EPW)])

    def sbody(i, c):
        ds_ = pl.ds(i * 16, 16)
        v = acc_v[ds_]
        plsc.addupdate_scatter(e2n_v, [dst_v[ds_]], v)
        plsc.addupdate_scatter(e2g_v, [eidx_v[ds_]], v)
        return c
    lax.fori_loop(0, nch, sbody, 0)

    pltpu.sync_copy(e2n_v, e2np.at[wid])
    pltpu.sync_copy(e2g_v, e2gp.at[wid])


def _c_body(x_ref, gT_ref, parts_ref, nrow_ref,
            wnx_ref, bn_ref, wng_ref, wcol_ref, lng_ref, lnb_ref, wn2_ref,
            nout_ref, n2g_ref):
    h = lax.dot_general(wnx_ref[...], x_ref[...], TRT, precision=HI,
                        preferred_element_type=F32) + bn_ref[...]
    gnT = jnp.dot(wng_ref[...], gT_ref[...], precision=HI, preferred_element_type=F32)
    ohg = (lax.broadcasted_iota(I32, (G, N), 0).astype(F32)
           == nrow_ref[...]).astype(F32)
    h = h + jnp.dot(gnT, ohg, precision=HI, preferred_element_type=F32)
    e2n_row = jnp.sum(parts_ref[...], axis=0, keepdims=True)
    h = h + wcol_ref[...] * e2n_row
    mu = jnp.mean(h, axis=0, keepdims=True)
    d0 = h - mu
    var = jnp.mean(d0 * d0, axis=0, keepdims=True)
    y = d0 * lax.rsqrt(var + 1e-5) * lng_ref[...] + lnb_ref[...]
    r = jnp.maximum(y, 0.0)
    r1 = jnp.concatenate([r, jnp.ones((1, N), F32)], axis=0)
    no = jnp.dot(wn2_ref[...], r1, precision=HI, preferred_element_type=F32)
    nout_ref[...] = no
    n2g_ref[...] = lax.dot_general(no, ohg, TRT, precision=HI, preferred_element_type=F32)


def _d_body(gT_ref, n2g_ref, e2gp_ref, wg1_ref, wg2_ref, wg3_ref, out_ref):
    e2g_row = jnp.sum(e2gp_ref[...], axis=0, keepdims=True)
    ones = jnp.ones((1, G), F32)
    gin = jnp.concatenate([gT_ref[...], n2g_ref[...], e2g_row, ones], axis=0)
    h1 = jnp.maximum(jnp.dot(wg1_ref[...], gin, preferred_element_type=F32), 0.0)
    h1 = jnp.concatenate([h1, ones], axis=0)
    h2 = jnp.maximum(jnp.dot(wg2_ref[...], h1, preferred_element_type=F32), 0.0)
    h2 = jnp.concatenate([h2, ones], axis=0)
    out_ref[...] = jnp.dot(wg3_ref[...], h2, preferred_element_type=F32)


@jax.jit
def kernel(x, e, g, edges, node_idx, edge_idx, W_e1, b_e1, W_e2, b_e2,
           W_n1, b_n1, ln_g, ln_b, W_n2, b_n2,
           W_g1, b_g1, W_g2, b_g2, W_g3, b_g3):
    src = edges[0]
    dst = edges[1]

    eT = e.T
    gT = g.T
    nrow = node_idx.astype(F32)[None, :]

    WE = jnp.zeros((16, 16), F32).at[:H].set(W_e1[:16].T)
    WA2 = jnp.concatenate([W_e1[16:144].T, W_e1[144:272].T], axis=0)
    WA2 = jnp.pad(WA2, ((0, 4), (0, 0)))
    WNX = W_n1[:128].T
    bn_col = b_n1[:, None]
    WNG = W_n1[128:160].T
    wcol = W_n1[160][:, None]
    lng = ln_g[:, None]
    lnb = ln_b[:, None]
    WN2 = jnp.concatenate([W_n2.T, b_n2[None, :]], axis=1)
    WG1 = jnp.concatenate([W_g1.T, b_g1[:, None]], axis=1)
    WG2 = jnp.concatenate([W_g2.T, b_g2[:, None]], axis=1)
    WG3 = jnp.concatenate([W_g3.T, b_g3[:, None]], axis=1)
    wtab = jnp.zeros((32,), F32).at[:H].set(W_e2[:, 0]).at[H].set(b_e2[0])
    wtab = wtab.at[16:16 + H].set(b_e1)
    wtab = jnp.tile(wtab[:, None], (1, 16))
    wgb = jnp.tile(W_e1[272:304].T.reshape(-1)[:, None], (1, 16))
    gtf = gT.reshape(-1)

    baseT = pl.pallas_call(
        _a1_body,
        grid=(E // BE,),
        in_specs=[
            pl.BlockSpec((16, BE), lambda i: (0, i)),
            pl.BlockSpec((16, 16), lambda i: (0, 0)),
        ],
        out_specs=pl.BlockSpec((16, BE), lambda i: (0, i)),
        out_shape=jax.ShapeDtypeStruct((16, E), F32),
    )(eT, WE)

    xallT = pl.pallas_call(
        _a2_body,
        grid=(1,),
        in_specs=[
            pl.BlockSpec((N, 128), lambda i: (0, 0)),
            pl.BlockSpec((24, 128), lambda i: (0, 0)),
        ],
        out_specs=pl.BlockSpec((24, N), lambda i: (0, 0)),
        out_shape=jax.ShapeDtypeStruct((24, N), F32),
    )(x, WA2)

    mesh = plsc.VectorSubcoreMesh(core_axis_name="c", subcore_axis_name="s")
    scb = functools.partial(
        pl.kernel,
        out_type=[
            jax.ShapeDtypeStruct((E,), F32),
            jax.ShapeDtypeStruct((NW, N), F32),
            jax.ShapeDtypeStruct((NW, G), F32),
        ],
        mesh=mesh,
        compiler_params=pltpu.CompilerParams(
            use_tc_tiling_on_sc=False, needs_layout_passes=False),
        scratch_types=[
            pltpu.VMEM((EPW,), I32),
            pltpu.VMEM((EPW,), I32),
            pltpu.VMEM((EPW,), I32),
            pltpu.VMEM((EPW,), F32),
            pltpu.VMEM((EPW,), F32),
            pltpu.VMEM((EPW,), F32),
            pltpu.VMEM((N,), F32),
            pltpu.VMEM((N,), F32),
            pltpu.VMEM((N,), F32),
            pltpu.VMEM((N,), F32),
            pltpu.VMEM((N,), F32),
            pltpu.VMEM((G,), F32),
            pltpu.VMEM((32, 16), F32),
            pltpu.VMEM((320, 16), F32),
            pltpu.VMEM((2048,), F32),
            pltpu.VMEM((H * G,), F32),
            pltpu.SemaphoreType.DMA,
            pltpu.SemaphoreType.DMA,
        ],
    )
    edge_out_flat, e2n_parts, e2g_parts = scb(_b_body)(
        baseT, xallT, src, dst, edge_idx, wtab, wgb, gtf)

    node_outT, n2g = pl.pallas_call(
        _c_body,
        grid=(1,),
        in_specs=[
            pl.BlockSpec((N, 128), lambda i: (0, 0)),
            pl.BlockSpec((32, G), lambda i: (0, 0)),
            pl.BlockSpec((NW, N), lambda i: (0, 0)),
            pl.BlockSpec((1, N), lambda i: (0, 0)),
            pl.BlockSpec((H, 128), lambda i: (0, 0)),
            pl.BlockSpec((H, 1), lambda i: (0, 0)),
            pl.BlockSpec((H, 32), lambda i: (0, 0)),
            pl.BlockSpec((H, 1), lambda i: (0, 0)),
            pl.BlockSpec((H, 1), lambda i: (0, 0)),
            pl.BlockSpec((H, 1), lambda i: (0, 0)),
            pl.BlockSpec((1, H + 1), lambda i: (0, 0)),
        ],
        out_specs=[
            pl.BlockSpec((1, N), lambda i: (0, 0)),
            pl.BlockSpec((1, G), lambda i: (0, 0)),
        ],
        out_shape=[
            jax.ShapeDtypeStruct((1, N), F32),
            jax.ShapeDtypeStruct((1, G), F32),
        ],
    )(x, gT, e2n_parts, nrow, WNX, bn_col, WNG, wcol, lng, lnb, WN2)

    globT = pl.pallas_call(
        _d_body,
        grid=(1,),
        in_specs=[
            pl.BlockSpec((32, G), lambda i: (0, 0)),
            pl.BlockSpec((1, G), lambda i: (0, 0)),
            pl.BlockSpec((NW, G), lambda i: (0, 0)),
            pl.BlockSpec((H, 35), lambda i: (0, 0)),
            pl.BlockSpec((H, H + 1), lambda i: (0, 0)),
            pl.BlockSpec((1, H + 1), lambda i: (0, 0)),
        ],
        out_specs=pl.BlockSpec((1, G), lambda i: (0, 0)),
        out_shape=jax.ShapeDtypeStruct((1, G), F32),
    )(gT, n2g, e2g_parts, WG1, WG2, WG3)

    edge_out = edge_out_flat[:, None]
    node_out = node_outT.reshape(N, 1)
    glob_out = globT.reshape(G, 1)
    return (edge_out, node_out, glob_out)


# speedup vs baseline: 2.2648x; 1.2400x over previous
import functools
import jax
import jax.numpy as jnp
from jax import lax
from jax.experimental import pallas as pl
from jax.experimental.pallas import tpu as pltpu
from jax.experimental.pallas import tpu_sc as plsc

N = 10000
E = 320000
G = 64
H = 10
NW = 32
EPW = E // NW
BE = 6400
BN = 2000
F32 = jnp.float32
I32 = jnp.int32
TRT = (((1,), (1,)), ((), ()))
HI = lax.Precision.HIGHEST


def _a1_body(eT_ref, w_ref, out_ref):
    out_ref[...] = jnp.dot(w_ref[...], eT_ref[...], preferred_element_type=F32)


def _a2_body(x_ref, w_ref, out_ref):
    out_ref[...] = lax.dot_general(w_ref[...], x_ref[...], TRT, precision=HI,
                                   preferred_element_type=F32)


def _b_body(baseT, xallT, srch, dsth, eidxh, wth, wgbh, gtfh,
            eo, e2np, e2gp,
            src_v, dst_v, eidx_v, acc_v,
            base0, base1, xs0, xs1, xd0, xd1,
            e2n_v, e2g_v, wt_v, wgb_v, gt_v, gp_v, sem0, sem1):
    wid = lax.axis_index("s") * 2 + lax.axis_index("c")
    be = wid * EPW
    pltpu.sync_copy(srch.at[pl.ds(be, EPW)], src_v)
    pltpu.sync_copy(dsth.at[pl.ds(be, EPW)], dst_v)
    pltpu.sync_copy(eidxh.at[pl.ds(be, EPW)], eidx_v)
    pltpu.sync_copy(wth, wt_v)
    pltpu.sync_copy(wgbh, wgb_v)
    pltpu.sync_copy(gtfh, gt_v)
    nch = EPW // 16

    def gpbody(d, c):
        for cc in range(G // 16):
            a = wt_v[16 + d, :]
            for k in range(32):
                a = a + gt_v[pl.ds(k * G + cc * 16, 16)] * wgb_v[d * 32 + k, :]
            gp_v[pl.ds(d * G + cc * 16, 16)] = a
        return c
    lax.fori_loop(0, H, gpbody, 0)

    b_vec = wt_v[H, :]

    @plsc.parallel_loop(0, nch, unroll=8)
    def zacc(i):
        acc_v[pl.ds(i * 16, 16)] = b_vec

    zero = jnp.zeros((16,), F32)

    @plsc.parallel_loop(0, N // 16, unroll=8)
    def zn(i):
        e2n_v[pl.ds(i * 16, 16)] = zero

    for j in range(G // 16):
        e2g_v[pl.ds(j * 16, 16)] = zero

    bufs = [(base0, xs0, xd0, sem0), (base1, xs1, xd1, sem1)]

    def start(d):
        bb, b1, b2, sm = bufs[d % 2]
        return [pltpu.async_copy(baseT.at[d, pl.ds(be, EPW)], bb, sm),
                pltpu.async_copy(xallT.at[d], b1, sm),
                pltpu.async_copy(xallT.at[H + d], b2, sm)]

    cps = start(0)
    for d in range(H):
        for c in cps:
            c.wait()
        if d < H - 1:
            cps = start(d + 1)
        base_v, xs_v, xd_v, _ = bufs[d % 2]
        w_vec = wt_v[d, :]
        off = jnp.full((16,), d * G, I32)

        @plsc.parallel_loop(0, nch, unroll=4)
        def ebody(i):
            ds_ = pl.ds(i * 16, 16)
            hh = (base_v[ds_]
                  + plsc.load_gather(xs_v, [src_v[ds_]])
                  + plsc.load_gather(xd_v, [dst_v[ds_]])
                  + plsc.load_gather(gp_v, [eidx_v[ds_] + off]))
            acc_v[ds_] = acc_v[ds_] + jnp.maximum(hh, 0.0) * w_vec

    pltpu.sync_copy(acc_v, eo.at[pl.ds(be, EPW)])

    def sbody(i, c):
        ds_ = pl.ds(i * 16, 16)
        v = acc_v[ds_]
        plsc.addupdate_scatter(e2n_v, [dst_v[ds_]], v)
        plsc.addupdate_scatter(e2g_v, [eidx_v[ds_]], v)
        return c
    lax.fori_loop(0, nch, sbody, 0)

    pltpu.sync_copy(e2n_v, e2np.at[wid])
    pltpu.sync_copy(e2g_v, e2gp.at[wid])


def _c_body(x_ref, gT_ref, parts_ref, nrow_ref,
            wnx_ref, bn_ref, wng_ref, wcol_ref, lng_ref, lnb_ref, wn2_ref,
            nout_ref, n2g_ref):
    h = lax.dot_general(wnx_ref[...], x_ref[...], TRT, precision=HI,
                        preferred_element_type=F32) + bn_ref[...]
    gnT = jnp.dot(wng_ref[...], gT_ref[...], precision=HI, preferred_element_type=F32)
    ohg = (lax.broadcasted_iota(I32, (G, N), 0).astype(F32)
           == nrow_ref[...]).astype(F32)
    h = h + jnp.dot(gnT, ohg, precision=HI, preferred_element_type=F32)
    e2n_row = jnp.sum(parts_ref[...], axis=0, keepdims=True)
    h = h + wcol_ref[...] * e2n_row
    mu = jnp.mean(h, axis=0, keepdims=True)
    d0 = h - mu
    var = jnp.mean(d0 * d0, axis=0, keepdims=True)
    y = d0 * lax.rsqrt(var + 1e-5) * lng_ref[...] + lnb_ref[...]
    r = jnp.maximum(y, 0.0)
    r1 = jnp.concatenate([r, jnp.ones((1, N), F32)], axis=0)
    no = jnp.dot(wn2_ref[...], r1, precision=HI, preferred_element_type=F32)
    nout_ref[...] = no
    n2g_ref[...] = lax.dot_general(no, ohg, TRT, precision=HI, preferred_element_type=F32)


def _d_body(gT_ref, n2g_ref, e2gp_ref, wg1_ref, wg2_ref, wg3_ref, out_ref):
    e2g_row = jnp.sum(e2gp_ref[...], axis=0, keepdims=True)
    ones = jnp.ones((1, G), F32)
    gin = jnp.concatenate([gT_ref[...], n2g_ref[...], e2g_row, ones], axis=0)
    h1 = jnp.maximum(jnp.dot(wg1_ref[...], gin, preferred_element_type=F32), 0.0)
    h1 = jnp.concatenate([h1, ones], axis=0)
    h2 = jnp.maximum(jnp.dot(wg2_ref[...], h1, preferred_element_type=F32), 0.0)
    h2 = jnp.concatenate([h2, ones], axis=0)
    out_ref[...] = jnp.dot(wg3_ref[...], h2, preferred_element_type=F32)


@jax.jit
def kernel(x, e, g, edges, node_idx, edge_idx, W_e1, b_e1, W_e2, b_e2,
           W_n1, b_n1, ln_g, ln_b, W_n2, b_n2,
           W_g1, b_g1, W_g2, b_g2, W_g3, b_g3):
    src = edges[0]
    dst = edges[1]

    eT = e.T
    gT = g.T
    nrow = node_idx.astype(F32)[None, :]

    WE = jnp.zeros((16, 16), F32).at[:H].set(W_e1[:16].T)
    WA2 = jnp.concatenate([W_e1[16:144].T, W_e1[144:272].T], axis=0)
    WA2 = jnp.pad(WA2, ((0, 4), (0, 0)))
    WNX = W_n1[:128].T
    bn_col = b_n1[:, None]
    WNG = W_n1[128:160].T
    wcol = W_n1[160][:, None]
    lng = ln_g[:, None]
    lnb = ln_b[:, None]
    WN2 = jnp.concatenate([W_n2.T, b_n2[None, :]], axis=1)
    WG1 = jnp.concatenate([W_g1.T, b_g1[:, None]], axis=1)
    WG2 = jnp.concatenate([W_g2.T, b_g2[:, None]], axis=1)
    WG3 = jnp.concatenate([W_g3.T, b_g3[:, None]], axis=1)
    wtab = jnp.zeros((32,), F32).at[:H].set(W_e2[:, 0]).at[H].set(b_e2[0])
    wtab = wtab.at[16:16 + H].set(b_e1)
    wtab = jnp.tile(wtab[:, None], (1, 16))
    wgb = jnp.tile(W_e1[272:304].T.reshape(-1)[:, None], (1, 16))
    gtf = gT.reshape(-1)

    baseT = pl.pallas_call(
        _a1_body,
        grid=(E // BE,),
        in_specs=[
            pl.BlockSpec((16, BE), lambda i: (0, i)),
            pl.BlockSpec((16, 16), lambda i: (0, 0)),
        ],
        out_specs=pl.BlockSpec((16, BE), lambda i: (0, i)),
        out_shape=jax.ShapeDtypeStruct((16, E), F32),
    )(eT, WE)

    xallT = pl.pallas_call(
        _a2_body,
        grid=(1,),
        in_specs=[
            pl.BlockSpec((N, 128), lambda i: (0, 0)),
            pl.BlockSpec((24, 128), lambda i: (0, 0)),
        ],
        out_specs=pl.BlockSpec((24, N), lambda i: (0, 0)),
        out_shape=jax.ShapeDtypeStruct((24, N), F32),
    )(x, WA2)

    mesh = plsc.VectorSubcoreMesh(core_axis_name="c", subcore_axis_name="s")
    scb = functools.partial(
        pl.kernel,
        out_type=[
            jax.ShapeDtypeStruct((E,), F32),
            jax.ShapeDtypeStruct((NW, N), F32),
            jax.ShapeDtypeStruct((NW, G), F32),
        ],
        mesh=mesh,
        compiler_params=pltpu.CompilerParams(
            use_tc_tiling_on_sc=False, needs_layout_passes=False),
        scratch_types=[
            pltpu.VMEM((EPW,), I32),
            pltpu.VMEM((EPW,), I32),
            pltpu.VMEM((EPW,), I32),
            pltpu.VMEM((EPW,), F32),
            pltpu.VMEM((EPW,), F32),
            pltpu.VMEM((EPW,), F32),
            pltpu.VMEM((N,), F32),
            pltpu.VMEM((N,), F32),
            pltpu.VMEM((N,), F32),
            pltpu.VMEM((N,), F32),
            pltpu.VMEM((N,), F32),
            pltpu.VMEM((G,), F32),
            pltpu.VMEM((32, 16), F32),
            pltpu.VMEM((320, 16), F32),
            pltpu.VMEM((2048,), F32),
            pltpu.VMEM((H * G,), F32),
            pltpu.SemaphoreType.DMA,
            pltpu.SemaphoreType.DMA,
        ],
    )
    edge_out_flat, e2n_parts, e2g_parts = scb(_b_body)(
        baseT, xallT, src, dst, edge_idx, wtab, wgb, gtf)

    node_outT, n2g = pl.pallas_call(
        _c_body,
        grid=(1,),
        in_specs=[
            pl.BlockSpec((N, 128), lambda i: (0, 0)),
            pl.BlockSpec((32, G), lambda i: (0, 0)),
            pl.BlockSpec((NW, N), lambda i: (0, 0)),
            pl.BlockSpec((1, N), lambda i: (0, 0)),
            pl.BlockSpec((H, 128), lambda i: (0, 0)),
            pl.BlockSpec((H, 1), lambda i: (0, 0)),
            pl.BlockSpec((H, 32), lambda i: (0, 0)),
            pl.BlockSpec((H, 1), lambda i: (0, 0)),
            pl.BlockSpec((H, 1), lambda i: (0, 0)),
            pl.BlockSpec((H, 1), lambda i: (0, 0)),
            pl.BlockSpec((1, H + 1), lambda i: (0, 0)),
        ],
        out_specs=[
            pl.BlockSpec((1, N), lambda i: (0, 0)),
            pl.BlockSpec((1, G), lambda i: (0, 0)),
        ],
        out_shape=[
            jax.ShapeDtypeStruct((1, N), F32),
            jax.ShapeDtypeStruct((1, G), F32),
        ],
    )(x, gT, e2n_parts, nrow, WNX, bn_col, WNG, wcol, lng, lnb, WN2)

    globT = pl.pallas_call(
        _d_body,
        grid=(1,),
        in_specs=[
            pl.BlockSpec((32, G), lambda i: (0, 0)),
            pl.BlockSpec((1, G), lambda i: (0, 0)),
            pl.BlockSpec((NW, G), lambda i: (0, 0)),
            pl.BlockSpec((H, 35), lambda i: (0, 0)),
            pl.BlockSpec((H, H + 1), lambda i: (0, 0)),
            pl.BlockSpec((1, H + 1), lambda i: (0, 0)),
        ],
        out_specs=pl.BlockSpec((1, G), lambda i: (0, 0)),
        out_shape=jax.ShapeDtypeStruct((1, G), F32),
    )(gT, n2g, e2g_parts, WG1, WG2, WG3)

    edge_out = edge_out_flat[:, None]
    node_out = node_outT.reshape(N, 1)
    glob_out = globT.reshape(G, 1)
    return (edge_out, node_out, glob_out)


# speedup vs baseline: 2.2923x; 1.0121x over previous
import functools
import jax
import jax.numpy as jnp
from jax import lax
from jax.experimental import pallas as pl
from jax.experimental.pallas import tpu as pltpu
from jax.experimental.pallas import tpu_sc as plsc

N = 10000
E = 320000
G = 64
H = 10
NW = 32
EPW = E // NW
BE = 6400
BN = 2000
F32 = jnp.float32
I32 = jnp.int32
TRT = (((1,), (1,)), ((), ()))
HI = lax.Precision.HIGHEST
BF = jnp.bfloat16


def _bf16r(v):
    i = lax.bitcast_convert_type(v, I32)
    i = i + jnp.int32(0x7FFF) + lax.shift_right_logical(i, 16).astype(I32) % 2
    i = lax.bitwise_and(i, jnp.int32(-65536))
    return lax.bitcast_convert_type(i, F32)


def _a1_body(eT_ref, gT_ref, w_ref, wg_ref, be1_ref, out_ref, gp_ref):
    out_ref[...] = jnp.dot(w_ref[...], eT_ref[...],
                           preferred_element_type=F32)

    @pl.when(pl.program_id(0) == 0)
    def _():
        gp_ref[...] = (jnp.dot(wg_ref[...], gT_ref[...],
                               preferred_element_type=F32) + be1_ref[...])


def _a2_body(x_ref, w_ref, out_ref):
    out_ref[...] = lax.dot_general(w_ref[...], x_ref[...], TRT,
                                   preferred_element_type=F32)


def _b_body(baseT, xallT, srch, dsth, eidxh, wth, gpfh,
            eo, e2np, e2gp,
            src_v, dst_v, eidx_v, acc_v,
            base0, base1, xs0, xs1, xd0, xd1,
            e2n_v, e2g_v, wt_v, gp_v, sem0, sem1):
    wid = lax.axis_index("s") * 2 + lax.axis_index("c")
    be = wid * EPW
    pltpu.sync_copy(srch.at[pl.ds(be, EPW)], src_v)
    pltpu.sync_copy(dsth.at[pl.ds(be, EPW)], dst_v)
    pltpu.sync_copy(eidxh.at[pl.ds(be, EPW)], eidx_v)
    pltpu.sync_copy(wth, wt_v)
    pltpu.sync_copy(gpfh, gp_v)
    nch = EPW // 16

    b_vec = wt_v[H, :]

    @plsc.parallel_loop(0, nch, unroll=8)
    def zacc(i):
        acc_v[pl.ds(i * 16, 16)] = b_vec

    zero = jnp.zeros((16,), F32)

    @plsc.parallel_loop(0, N // 16, unroll=8)
    def zn(i):
        e2n_v[pl.ds(i * 16, 16)] = zero

    for j in range(G // 16):
        e2g_v[pl.ds(j * 16, 16)] = zero

    bufs = [(base0, xs0, xd0, sem0), (base1, xs1, xd1, sem1)]

    def start(d):
        bb, b1, b2, sm = bufs[d % 2]
        return [pltpu.async_copy(baseT.at[d, pl.ds(be, EPW)], bb, sm),
                pltpu.async_copy(xallT.at[d], b1, sm),
                pltpu.async_copy(xallT.at[H + d], b2, sm)]

    cps = start(0)
    for d in range(H):
        for c in cps:
            c.wait()
        if d < H - 1:
            cps = start(d + 1)
        base_v, xs_v, xd_v, _ = bufs[d % 2]
        w_vec = wt_v[d, :]
        off = jnp.full((16,), d * G, I32)

        @plsc.parallel_loop(0, nch, unroll=4)
        def ebody(i):
            ds_ = pl.ds(i * 16, 16)
            hh = (base_v[ds_]
                  + plsc.load_gather(xs_v, [src_v[ds_]])
                  + plsc.load_gather(xd_v, [dst_v[ds_]])
                  + plsc.load_gather(gp_v, [eidx_v[ds_] + off]))
            acc_v[ds_] = acc_v[ds_] + _bf16r(jnp.maximum(hh, 0.0)) * w_vec

    pltpu.sync_copy(acc_v, eo.at[pl.ds(be, EPW)])

    def sbody(i, c):
        ds_ = pl.ds(i * 16, 16)
        v = acc_v[ds_]
        plsc.addupdate_scatter(e2n_v, [dst_v[ds_]], v)
        plsc.addupdate_scatter(e2g_v, [eidx_v[ds_]], v)
        return c
    lax.fori_loop(0, nch, sbody, 0)

    pltpu.sync_copy(e2n_v, e2np.at[wid])
    pltpu.sync_copy(e2g_v, e2gp.at[wid])


def _c_body(x_ref, gT_ref, parts_ref, nrow_ref,
            wnx_ref, bn_ref, wng_ref, wcol_ref, lng_ref, lnb_ref, wn2_ref,
            nout_ref, n2g_ref):
    h = lax.dot_general(wnx_ref[...], x_ref[...], TRT,
                        preferred_element_type=F32) + bn_ref[...]
    gnT = jnp.dot(wng_ref[...], gT_ref[...], preferred_element_type=F32)
    ohg = (lax.broadcasted_iota(I32, (G, N), 0).astype(F32)
           == nrow_ref[...]).astype(F32)
    h = h + jnp.dot(gnT, ohg, precision=HI, preferred_element_type=F32)
    e2n_row = jnp.sum(parts_ref[...], axis=0, keepdims=True)
    e2n_row = e2n_row.astype(BF).astype(F32)
    h = h + wcol_ref[...] * e2n_row
    mu = jnp.mean(h, axis=0, keepdims=True)
    d0 = h - mu
    var = jnp.mean(d0 * d0, axis=0, keepdims=True)
    y = d0 * lax.rsqrt(var + 1e-5) * lng_ref[...] + lnb_ref[...]
    r = jnp.maximum(y, 0.0)
    r1 = jnp.concatenate([r, jnp.ones((1, N), F32)], axis=0).astype(BF)
    no = jnp.dot(wn2_ref[...], r1, preferred_element_type=F32)
    nout_ref[...] = no
    n2g_ref[...] = lax.dot_general(no, ohg, TRT, precision=HI,
                                   preferred_element_type=F32)


def _d_body(gT_ref, n2g_ref, e2gp_ref, wg1_ref, wg2_ref, wg3_ref, out_ref):
    e2g_row = jnp.sum(e2gp_ref[...], axis=0, keepdims=True)
    ones = jnp.ones((1, G), F32)
    gin = jnp.concatenate([gT_ref[...], n2g_ref[...], e2g_row, ones],
                          axis=0).astype(BF)
    h1 = jnp.maximum(jnp.dot(wg1_ref[...], gin, preferred_element_type=F32), 0.0)
    h1 = jnp.concatenate([h1, ones], axis=0).astype(BF)
    h2 = jnp.maximum(jnp.dot(wg2_ref[...], h1, preferred_element_type=F32), 0.0)
    h2 = jnp.concatenate([h2, ones], axis=0).astype(BF)
    out_ref[...] = jnp.dot(wg3_ref[...], h2, preferred_element_type=F32)


@jax.jit
def kernel(x, e, g, edges, node_idx, edge_idx, W_e1, b_e1, W_e2, b_e2,
           W_n1, b_n1, ln_g, ln_b, W_n2, b_n2,
           W_g1, b_g1, W_g2, b_g2, W_g3, b_g3):
    src = edges[0]
    dst = edges[1]

    eT = e.T.astype(BF)
    x_bf = x.astype(BF)
    gT = g.T
    gT_bf = gT.astype(BF)
    nrow = node_idx.astype(F32)[None, :]

    WE = jnp.zeros((16, 16), F32).at[:H].set(W_e1[:16].T).astype(BF)
    WA2 = jnp.concatenate([W_e1[16:144].T, W_e1[144:272].T], axis=0)
    WA2 = jnp.pad(WA2, ((0, 4), (0, 0))).astype(BF)
    WNX = W_n1[:128].T.astype(BF)
    bn_col = b_n1[:, None]
    WNG = W_n1[128:160].T.astype(BF)
    wcol = W_n1[160].astype(BF).astype(F32)[:, None]
    lng = ln_g[:, None]
    lnb = ln_b[:, None]
    WN2 = jnp.concatenate([W_n2.T, b_n2[None, :]], axis=1).astype(BF)
    WG1 = jnp.concatenate([W_g1.T, b_g1[:, None]], axis=1).astype(BF)
    WG2 = jnp.concatenate([W_g2.T, b_g2[:, None]], axis=1).astype(BF)
    WG3 = jnp.concatenate([W_g3.T, b_g3[:, None]], axis=1).astype(BF)
    wtab = jnp.zeros((32,), F32).at[:H].set(
        W_e2[:, 0].astype(BF).astype(F32)).at[H].set(b_e2[0])
    wtab = jnp.tile(wtab[:, None], (1, 16))
    WG16 = jnp.zeros((16, 32), F32).at[:H].set(W_e1[272:304].T).astype(BF)
    be1c = jnp.zeros((16, 1), F32).at[:H, 0].set(b_e1)

    baseT, gpb = pl.pallas_call(
        _a1_body,
        grid=(E // BE,),
        in_specs=[
            pl.BlockSpec((16, BE), lambda i: (0, i)),
            pl.BlockSpec((32, G), lambda i: (0, 0)),
            pl.BlockSpec((16, 16), lambda i: (0, 0)),
            pl.BlockSpec((16, 32), lambda i: (0, 0)),
            pl.BlockSpec((16, 1), lambda i: (0, 0)),
        ],
        out_specs=[
            pl.BlockSpec((16, BE), lambda i: (0, i)),
            pl.BlockSpec((16, G), lambda i: (0, 0)),
        ],
        out_shape=[
            jax.ShapeDtypeStruct((16, E), F32),
            jax.ShapeDtypeStruct((16, G), F32),
        ],
    )(eT, gT_bf, WE, WG16, be1c)
    gpf = gpb.reshape(-1)

    xallT = pl.pallas_call(
        _a2_body,
        grid=(1,),
        in_specs=[
            pl.BlockSpec((N, 128), lambda i: (0, 0)),
            pl.BlockSpec((24, 128), lambda i: (0, 0)),
        ],
        out_specs=pl.BlockSpec((24, N), lambda i: (0, 0)),
        out_shape=jax.ShapeDtypeStruct((24, N), F32),
    )(x_bf, WA2)

    mesh = plsc.VectorSubcoreMesh(core_axis_name="c", subcore_axis_name="s")
    scb = functools.partial(
        pl.kernel,
        out_type=[
            jax.ShapeDtypeStruct((E,), F32),
            jax.ShapeDtypeStruct((NW, N), F32),
            jax.ShapeDtypeStruct((NW, G), F32),
        ],
        mesh=mesh,
        compiler_params=pltpu.CompilerParams(
            use_tc_tiling_on_sc=False, needs_layout_passes=False),
        scratch_types=[
            pltpu.VMEM((EPW,), I32),
            pltpu.VMEM((EPW,), I32),
            pltpu.VMEM((EPW,), I32),
            pltpu.VMEM((EPW,), F32),
            pltpu.VMEM((EPW,), F32),
            pltpu.VMEM((EPW,), F32),
            pltpu.VMEM((N,), F32),
            pltpu.VMEM((N,), F32),
            pltpu.VMEM((N,), F32),
            pltpu.VMEM((N,), F32),
            pltpu.VMEM((N,), F32),
            pltpu.VMEM((G,), F32),
            pltpu.VMEM((32, 16), F32),
            pltpu.VMEM((16 * G,), F32),
            pltpu.SemaphoreType.DMA,
            pltpu.SemaphoreType.DMA,
        ],
    )
    edge_out_flat, e2n_parts, e2g_parts = scb(_b_body)(
        baseT, xallT, src, dst, edge_idx, wtab, gpf)

    node_outT, n2g = pl.pallas_call(
        _c_body,
        grid=(1,),
        in_specs=[
            pl.BlockSpec((N, 128), lambda i: (0, 0)),
            pl.BlockSpec((32, G), lambda i: (0, 0)),
            pl.BlockSpec((NW, N), lambda i: (0, 0)),
            pl.BlockSpec((1, N), lambda i: (0, 0)),
            pl.BlockSpec((H, 128), lambda i: (0, 0)),
            pl.BlockSpec((H, 1), lambda i: (0, 0)),
            pl.BlockSpec((H, 32), lambda i: (0, 0)),
            pl.BlockSpec((H, 1), lambda i: (0, 0)),
            pl.BlockSpec((H, 1), lambda i: (0, 0)),
            pl.BlockSpec((H, 1), lambda i: (0, 0)),
            pl.BlockSpec((1, H + 1), lambda i: (0, 0)),
        ],
        out_specs=[
            pl.BlockSpec((1, N), lambda i: (0, 0)),
            pl.BlockSpec((1, G), lambda i: (0, 0)),
        ],
        out_shape=[
            jax.ShapeDtypeStruct((1, N), F32),
            jax.ShapeDtypeStruct((1, G), F32),
        ],
    )(x_bf, gT_bf, e2n_parts, nrow, WNX, bn_col, WNG, wcol, lng, lnb, WN2)

    globT = pl.pallas_call(
        _d_body,
        grid=(1,),
        in_specs=[
            pl.BlockSpec((32, G), lambda i: (0, 0)),
            pl.BlockSpec((1, G), lambda i: (0, 0)),
            pl.BlockSpec((NW, G), lambda i: (0, 0)),
            pl.BlockSpec((H, 35), lambda i: (0, 0)),
            pl.BlockSpec((H, H + 1), lambda i: (0, 0)),
            pl.BlockSpec((1, H + 1), lambda i: (0, 0)),
        ],
        out_specs=pl.BlockSpec((1, G), lambda i: (0, 0)),
        out_shape=jax.ShapeDtypeStruct((1, G), F32),
    )(gT, n2g, e2g_parts, WG1, WG2, WG3)

    edge_out = edge_out_flat[:, None]
    node_out = node_outT.reshape(N, 1)
    glob_out = globT.reshape(G, 1)
    return (edge_out, node_out, glob_out)
